# baseline (speedup 1.0000x reference)
import functools

import jax
import jax.numpy as jnp
from jax import lax
from jax.experimental import pallas as pl
from jax.experimental.pallas import tpu as pltpu
from jax.experimental.pallas import tpu_sc as plsc

_NUM_VERTS = 642
_NUM_FACES = 1280
_UV_MAP = 1001
_VP = 648

_NC = 2
_NS = 16
_NW = _NC * _NS
_L = 16
_PB = 2048


def _uv3d_rows(u, v):
    phi = (2.0 * jnp.pi) * (u - 0.5)
    theta = jnp.pi * (v - 0.5)
    ct = jnp.cos(theta)
    return ct * jnp.cos(phi), ct * jnp.sin(phi), jnp.sin(theta)


def _tca_body(uv_ref, uvv_ref, faces_ref, verts_ref,
              uv8_ref, idx_ref, vt3d_ref, faces_t_ref, verts_t_ref):
    ub = uv_ref[...]
    u8 = ub[:, 0:1].reshape(8, _PB // 8)
    v8 = ub[:, 1:2].reshape(8, _PB // 8)
    uv8_ref[...] = jnp.concatenate([u8[None], v8[None]], axis=0)[None]
    xy = jnp.clip(jnp.round(ub * 1000.0).astype(jnp.int32), 0, _UV_MAP - 1)
    flat = xy[:, 1:2] * _UV_MAP + xy[:, 0:1]
    idx_ref[...] = flat.reshape(_PB // 128, 128)

    @pl.when(pl.program_id(0) == 0)
    def _():
        t = jnp.transpose(uvv_ref[...], (1, 0))
        x, y, z = _uv3d_rows(t[0:1, :], t[1:2, :])
        r = jnp.concatenate([x, y, z], axis=0)
        vt3d_ref[...] = jnp.pad(r, ((0, 0), (0, _VP - _NUM_VERTS)))
        faces_t_ref[...] = jnp.transpose(faces_ref[...], (1, 0))
        verts_t_ref[...] = jnp.pad(
            jnp.transpose(verts_ref[...], (1, 0)),
            ((0, 0), (0, _VP - _NUM_VERTS)))


def _tca_call(uv, uv_verts, faces, verts):
    n = uv.shape[0]
    nblk = n // _PB
    return pl.pallas_call(
        _tca_body,
        grid=(nblk,),
        in_specs=[
            pl.BlockSpec((_PB, 2), lambda k: (k, 0)),
            pl.BlockSpec((_NUM_VERTS, 2), lambda k: (0, 0)),
            pl.BlockSpec((_NUM_FACES, 3), lambda k: (0, 0)),
            pl.BlockSpec((_NUM_VERTS, 3), lambda k: (0, 0)),
        ],
        out_specs=[
            pl.BlockSpec((1, 2, 8, _PB // 8), lambda k: (k, 0, 0, 0)),
            pl.BlockSpec((_PB // 128, 128), lambda k: (k, 0)),
            pl.BlockSpec((3, _VP), lambda k: (0, 0)),
            pl.BlockSpec((3, _NUM_FACES), lambda k: (0, 0)),
            pl.BlockSpec((3, _VP), lambda k: (0, 0)),
        ],
        out_shape=[
            jax.ShapeDtypeStruct((nblk, 2, 8, _PB // 8), jnp.float32),
            jax.ShapeDtypeStruct((n // 128, 128), jnp.int32),
            jax.ShapeDtypeStruct((3, _VP), jnp.float32),
            jax.ShapeDtypeStruct((3, _NUM_FACES), jnp.int32),
            jax.ShapeDtypeStruct((3, _VP), jnp.float32),
        ],
    )(uv, uv_verts, faces, verts)


def _sc_gather_call(idx_fat, face_inds_flat, faces, verts, vt3d, n):
    np_per = n // _NW
    rows_per = np_per // 128
    mesh = plsc.VectorSubcoreMesh(
        core_axis_name="c", subcore_axis_name="s",
        num_cores=_NC, num_subcores=_NS)

    @functools.partial(
        pl.kernel,
        out_type=[
            jax.ShapeDtypeStruct((_NW, 3, np_per), jnp.int32),
            jax.ShapeDtypeStruct((_NW, 9, 8, np_per // 8), jnp.float32),
            jax.ShapeDtypeStruct((_NW, 9, 8, np_per // 8), jnp.float32),
        ],
        mesh=mesh,
        compiler_params=pltpu.CompilerParams(needs_layout_passes=False),
        scratch_types=[
            pltpu.VMEM((rows_per, 128), jnp.int32),
            pltpu.VMEM((np_per,), jnp.int32),
            pltpu.VMEM((3, _NUM_FACES), jnp.int32),
            pltpu.VMEM((3, _VP), jnp.float32),
            pltpu.VMEM((3, _VP), jnp.float32),
            pltpu.VMEM((3, np_per), jnp.int32),
            pltpu.VMEM((9, 8, np_per // 8), jnp.float32),
            pltpu.VMEM((9, 8, np_per // 8), jnp.float32),
            pltpu.SemaphoreType.DMA,
        ],
    )
    def sc_kernel(idx_hbm, tab_hbm, faces_hbm, verts_hbm, vt3d_hbm,
                  fvi_hbm, fv_hbm, fvt_hbm,
                  idx_v, fi_v, faces_v, verts_v, vt3d_v,
                  fvi_o, fv_o, fvt_o, sem):
        wid = lax.axis_index("s") * _NC + lax.axis_index("c")
        pltpu.sync_copy(idx_hbm.at[pl.ds(wid * rows_per, rows_per)], idx_v)
        pltpu.sync_copy(faces_hbm, faces_v)
        pltpu.sync_copy(verts_hbm, verts_v)
        pltpu.sync_copy(vt3d_hbm, vt3d_v)

        copies = []
        for r in range(rows_per):
            copies.append(pltpu.async_copy(
                tab_hbm.at[idx_v.at[r]],
                fi_v.at[pl.ds(r * 128, 128)], sem))
        for cp in copies:
            cp.wait()

        col0 = jnp.zeros((_L,), jnp.int32)
        col1 = jnp.ones((_L,), jnp.int32)
        col2 = jnp.full((_L,), 2, jnp.int32)
        cols = (col0, col1, col2)

        lpg = (np_per // 8) // _L

        def body(i, carry):
            f = fi_v[pl.ds(i * _L, _L)]
            sidx = i // lpg
            l0 = (i % lpg) * _L
            for j in range(3):
                vj = plsc.load_gather(faces_v, [cols[j], f])
                fvi_o[j, pl.ds(i * _L, _L)] = vj
                for c in range(3):
                    val = plsc.load_gather(verts_v, [cols[c], vj])
                    fv_o[3 * j + c, sidx, pl.ds(l0, _L)] = val
                for c in range(3):
                    val = plsc.load_gather(vt3d_v, [cols[c], vj])
                    fvt_o[3 * j + c, sidx, pl.ds(l0, _L)] = val
            return carry

        lax.fori_loop(0, np_per // _L, body, 0)

        pltpu.sync_copy(fvi_o, fvi_hbm.at[wid])
        pltpu.sync_copy(fv_o, fv_hbm.at[wid])
        pltpu.sync_copy(fvt_o, fvt_hbm.at[wid])

    return sc_kernel(idx_fat, face_inds_flat, faces, verts, vt3d)


def _cross(a, b):
    ax, ay, az = a
    bx, by, bz = b
    return (ay * bz - az * by, az * bx - ax * bz, ax * by - ay * bx)


def _norm3(a):
    ax, ay, az = a
    return jnp.sqrt(ax * ax + ay * ay + az * az)


def _sub(a, b):
    return (a[0] - b[0], a[1] - b[1], a[2] - b[2])


def _tcb_body(nblk, fvi_ref, fv_ref, fvt_ref, uv8_ref, feat_ref,
              p3d_ref, vlf_ref, cnt_s):
    p = pl.program_id(1)

    fvt = fvt_ref[...].reshape(9, 8, _PB // 8)
    pa = (fvt[0], fvt[1], fvt[2])
    pb3 = (fvt[3], fvt[4], fvt[5])
    pc = (fvt[6], fvt[7], fvt[8])
    uvr = uv8_ref[...].reshape(2, 8, _PB // 8)
    pt = _uv3d_rows(uvr[0], uvr[1])
    ab = _sub(pb3, pa)
    ac = _sub(pc, pa)
    bc = _sub(pc, pb3)
    ap = _sub(pt, pa)
    bp = _sub(pt, pb3)
    area_bac = _norm3(_cross(ab, ac))
    area_bap = _norm3(_cross(ab, ap))
    area_cap = _norm3(_cross(ac, ap))
    area_cbp = _norm3(_cross(bc, bp))
    w = area_bap / area_bac
    v = area_cap / area_bac
    u = area_cbp / area_bac
    denom = jnp.maximum(jnp.abs(u) + jnp.abs(v) + jnp.abs(w), 1e-12)
    u = u / denom
    v = v / denom
    w = w / denom

    fv = fv_ref[...].reshape(9, 8, _PB // 8)
    px = u * fv[0] + v * fv[3] + w * fv[6]
    py = u * fv[1] + v * fv[4] + w * fv[7]
    pz = u * fv[2] + v * fv[5] + w * fv[8]
    p3d_ref[...] = jnp.concatenate(
        [px[None], py[None], pz[None]], axis=0)[None]

    fvi = fvi_ref[...].reshape(3, _PB)
    vid = lax.broadcasted_iota(jnp.int32, (_VP, _PB), 0)
    wmat = ((vid == fvi[0:1, :]).astype(jnp.float32)
            + (vid == fvi[1:2, :]).astype(jnp.float32)
            + (vid == fvi[2:3, :]).astype(jnp.float32)).astype(jnp.bfloat16)

    feat = feat_ref[...].reshape(_PB, -1).astype(jnp.bfloat16)
    part = lax.dot_general(wmat, feat, (((1,), (0,)), ((), ())),
                           preferred_element_type=jnp.float32)
    ones8 = jnp.ones((_PB, 8), jnp.bfloat16)
    cnt8 = lax.dot_general(wmat, ones8, (((1,), (0,)), ((), ())),
                           preferred_element_type=jnp.float32)
    pcb = jnp.broadcast_to(cnt8[:, 0:1], (_VP, 128))

    partv = part[:_NUM_VERTS]
    pcbv = pcb[:_NUM_VERTS]

    @pl.when(p == 0)
    def _():
        vlf_ref[...] = partv[None]
        cnt_s[...] = pcbv

    @pl.when(p != 0)
    def _():
        vlf_ref[...] = vlf_ref[...] + partv[None]
        cnt_s[...] = cnt_s[...] + pcbv

    @pl.when(p == nblk - 1)
    def _():
        cnt = jnp.maximum(cnt_s[:, 0:1], 1.0)
        vlf_ref[...] = vlf_ref[...] / cnt[None]


def _tcb_call(fvi, fverts, fvt3d, uvrows, local_feature):
    b, pdim, d = local_feature.shape
    nw, _, pb = fvi.shape
    n = nw * pb
    nblk = pdim // pb
    assert b * nblk == nw and pb == _PB
    grid = (b, nblk)
    body = functools.partial(_tcb_body, nblk)
    return pl.pallas_call(
        body,
        grid=grid,
        in_specs=[
            pl.BlockSpec((1, 3, pb), lambda bi, pi: (bi * nblk + pi, 0, 0)),
            pl.BlockSpec((1, 9, 8, pb // 8),
                         lambda bi, pi: (bi * nblk + pi, 0, 0, 0)),
            pl.BlockSpec((1, 9, 8, pb // 8),
                         lambda bi, pi: (bi * nblk + pi, 0, 0, 0)),
            pl.BlockSpec((1, 2, 8, pb // 8),
                         lambda bi, pi: (bi * nblk + pi, 0, 0, 0)),
            pl.BlockSpec((1, pb, d), lambda bi, pi: (bi, pi, 0)),
        ],
        out_specs=[
            pl.BlockSpec((1, 3, 8, pb // 8),
                         lambda bi, pi: (bi * nblk + pi, 0, 0, 0)),
            pl.BlockSpec((1, _NUM_VERTS, d), lambda bi, pi: (bi, 0, 0)),
        ],
        out_shape=[
            jax.ShapeDtypeStruct((nw, 3, 8, pb // 8), jnp.float32),
            jax.ShapeDtypeStruct((b, _NUM_VERTS, d), jnp.float32),
        ],
        scratch_shapes=[pltpu.VMEM((_NUM_VERTS, 128), jnp.float32)],
    )(fvi, fverts, fvt3d, uvrows, local_feature)


def kernel(uv, local_feature, verts, uv_verts, faces, face_inds):
    n = uv.shape[0]
    uv8, idx_fat, vt3d, faces_t, verts_t = _tca_call(
        uv, uv_verts, faces, verts)
    fvi, fverts, fvt3d = _sc_gather_call(
        idx_fat, face_inds.reshape(-1), faces_t, verts_t, vt3d, n)
    p3d4, vlf = _tcb_call(fvi, fverts, fvt3d, uv8, local_feature)
    p3d = jnp.transpose(p3d4, (0, 2, 3, 1)).reshape(n, 3)
    return p3d, vlf


# speedup vs baseline: 1.1230x; 1.1230x over previous
import functools

import jax
import jax.numpy as jnp
from jax import lax
from jax.experimental import pallas as pl
from jax.experimental.pallas import tpu as pltpu
from jax.experimental.pallas import tpu_sc as plsc

_NUM_VERTS = 642
_NUM_FACES = 1280
_UV_MAP = 1001
_VP = 648

_NC = 2
_NS = 16
_NW = _NC * _NS
_L = 16
_PB = 2048


def _uv3d_rows(u, v):
    phi = (2.0 * jnp.pi) * (u - 0.5)
    theta = jnp.pi * (v - 0.5)
    ct = jnp.cos(theta)
    return ct * jnp.cos(phi), ct * jnp.sin(phi), jnp.sin(theta)


def _tca_body(uv_ref, uvv_ref, faces_ref, verts_ref,
              uvrows_ref, idx_ref, vt3d_ref, faces_t_ref, verts_t_ref):
    uvt = jnp.transpose(uv_ref[...], (1, 0))
    uvrows_ref[...] = uvt[None]
    u = uvt[0:1, :]
    v = uvt[1:2, :]
    xi = jnp.clip(jnp.round(u * 1000.0).astype(jnp.int32), 0, _UV_MAP - 1)
    yi = jnp.clip(jnp.round(v * 1000.0).astype(jnp.int32), 0, _UV_MAP - 1)
    idx_ref[...] = (yi * _UV_MAP + xi).reshape(_PB // 128, 128)

    @pl.when(pl.program_id(0) == 0)
    def _():
        t = jnp.transpose(uvv_ref[...], (1, 0))
        x, y, z = _uv3d_rows(t[0:1, :], t[1:2, :])
        r = jnp.concatenate([x, y, z], axis=0)
        vt3d_ref[...] = jnp.pad(r, ((0, 0), (0, _VP - _NUM_VERTS)))
        faces_t_ref[...] = jnp.transpose(faces_ref[...], (1, 0))
        verts_t_ref[...] = jnp.pad(
            jnp.transpose(verts_ref[...], (1, 0)),
            ((0, 0), (0, _VP - _NUM_VERTS)))


def _tca_call(uv, uv_verts, faces, verts):
    n = uv.shape[0]
    nblk = n // _PB
    return pl.pallas_call(
        _tca_body,
        grid=(nblk,),
        in_specs=[
            pl.BlockSpec((_PB, 2), lambda k: (k, 0)),
            pl.BlockSpec((_NUM_VERTS, 2), lambda k: (0, 0)),
            pl.BlockSpec((_NUM_FACES, 3), lambda k: (0, 0)),
            pl.BlockSpec((_NUM_VERTS, 3), lambda k: (0, 0)),
        ],
        out_specs=[
            pl.BlockSpec((1, 2, _PB), lambda k: (k, 0, 0)),
            pl.BlockSpec((_PB // 128, 128), lambda k: (k, 0)),
            pl.BlockSpec((3, _VP), lambda k: (0, 0)),
            pl.BlockSpec((3, _NUM_FACES), lambda k: (0, 0)),
            pl.BlockSpec((3, _VP), lambda k: (0, 0)),
        ],
        out_shape=[
            jax.ShapeDtypeStruct((nblk, 2, _PB), jnp.float32),
            jax.ShapeDtypeStruct((n // 128, 128), jnp.int32),
            jax.ShapeDtypeStruct((3, _VP), jnp.float32),
            jax.ShapeDtypeStruct((3, _NUM_FACES), jnp.int32),
            jax.ShapeDtypeStruct((3, _VP), jnp.float32),
        ],
    )(uv, uv_verts, faces, verts)


def _sc_gather_call(idx_fat, uvrows, face_inds_flat, faces, verts, vt3d, n):
    np_per = n // _NW
    rows_per = np_per // 128
    mesh = plsc.VectorSubcoreMesh(
        core_axis_name="c", subcore_axis_name="s",
        num_cores=_NC, num_subcores=_NS)

    @functools.partial(
        pl.kernel,
        out_type=[
            jax.ShapeDtypeStruct((_NW, 3, np_per), jnp.int32),
            jax.ShapeDtypeStruct((_NW, 9, 8, np_per // 8), jnp.float32),
            jax.ShapeDtypeStruct((_NW, 9, 8, np_per // 8), jnp.float32),
            jax.ShapeDtypeStruct((_NW, 2, 8, np_per // 8), jnp.float32),
        ],
        mesh=mesh,
        compiler_params=pltpu.CompilerParams(needs_layout_passes=False),
        scratch_types=[
            pltpu.VMEM((rows_per, 128), jnp.int32),
            pltpu.VMEM((2, np_per), jnp.float32),
            pltpu.VMEM((np_per,), jnp.int32),
            pltpu.VMEM((3, _NUM_FACES), jnp.int32),
            pltpu.VMEM((3, _VP), jnp.float32),
            pltpu.VMEM((3, _VP), jnp.float32),
            pltpu.VMEM((3, np_per), jnp.int32),
            pltpu.VMEM((9, 8, np_per // 8), jnp.float32),
            pltpu.VMEM((9, 8, np_per // 8), jnp.float32),
            pltpu.VMEM((2, 8, np_per // 8), jnp.float32),
            pltpu.SemaphoreType.DMA,
        ],
    )
    def sc_kernel(idx_hbm, uvr_hbm, tab_hbm, faces_hbm, verts_hbm, vt3d_hbm,
                  fvi_hbm, fv_hbm, fvt_hbm, uv8_hbm,
                  idx_v, uvr_v, fi_v, faces_v, verts_v, vt3d_v,
                  fvi_o, fv_o, fvt_o, uv8_o, sem):
        wid = lax.axis_index("s") * _NC + lax.axis_index("c")
        pltpu.sync_copy(idx_hbm.at[pl.ds(wid * rows_per, rows_per)], idx_v)
        pltpu.sync_copy(uvr_hbm.at[wid], uvr_v)
        pltpu.sync_copy(faces_hbm, faces_v)
        pltpu.sync_copy(verts_hbm, verts_v)
        pltpu.sync_copy(vt3d_hbm, vt3d_v)

        copies = []
        for r in range(rows_per):
            copies.append(pltpu.async_copy(
                tab_hbm.at[idx_v.at[r]],
                fi_v.at[pl.ds(r * 128, 128)], sem))
        for cp in copies:
            cp.wait()

        col0 = jnp.zeros((_L,), jnp.int32)
        col1 = jnp.ones((_L,), jnp.int32)
        col2 = jnp.full((_L,), 2, jnp.int32)
        cols = (col0, col1, col2)

        lpg = (np_per // 8) // _L

        def body(i, carry):
            f = fi_v[pl.ds(i * _L, _L)]
            sidx = i // lpg
            l0 = (i % lpg) * _L
            uv8_o[0, sidx, pl.ds(l0, _L)] = uvr_v[0, pl.ds(i * _L, _L)]
            uv8_o[1, sidx, pl.ds(l0, _L)] = uvr_v[1, pl.ds(i * _L, _L)]
            for j in range(3):
                vj = plsc.load_gather(faces_v, [cols[j], f])
                fvi_o[j, pl.ds(i * _L, _L)] = vj
                for c in range(3):
                    val = plsc.load_gather(verts_v, [cols[c], vj])
                    fv_o[3 * j + c, sidx, pl.ds(l0, _L)] = val
                for c in range(3):
                    val = plsc.load_gather(vt3d_v, [cols[c], vj])
                    fvt_o[3 * j + c, sidx, pl.ds(l0, _L)] = val
            return carry

        lax.fori_loop(0, np_per // _L, body, 0)

        pltpu.sync_copy(fvi_o, fvi_hbm.at[wid])
        pltpu.sync_copy(fv_o, fv_hbm.at[wid])
        pltpu.sync_copy(fvt_o, fvt_hbm.at[wid])
        pltpu.sync_copy(uv8_o, uv8_hbm.at[wid])

    return sc_kernel(idx_fat, uvrows, face_inds_flat, faces, verts, vt3d)


def _cross(a, b):
    ax, ay, az = a
    bx, by, bz = b
    return (ay * bz - az * by, az * bx - ax * bz, ax * by - ay * bx)


def _norm3(a):
    ax, ay, az = a
    return jnp.sqrt(ax * ax + ay * ay + az * az)


def _sub(a, b):
    return (a[0] - b[0], a[1] - b[1], a[2] - b[2])


def _tcb_body(nblk, fvi_ref, fv_ref, fvt_ref, uv8_ref, feat_ref,
              p3d_ref, vlf_ref, cnt_s):
    p = pl.program_id(1)

    fvt = fvt_ref[...].reshape(9, 8, _PB // 8)
    pa = (fvt[0], fvt[1], fvt[2])
    pb3 = (fvt[3], fvt[4], fvt[5])
    pc = (fvt[6], fvt[7], fvt[8])
    uvr = uv8_ref[...].reshape(2, 8, _PB // 8)
    pt = _uv3d_rows(uvr[0], uvr[1])
    ab = _sub(pb3, pa)
    ac = _sub(pc, pa)
    bc = _sub(pc, pb3)
    ap = _sub(pt, pa)
    bp = _sub(pt, pb3)
    area_bac = _norm3(_cross(ab, ac))
    area_bap = _norm3(_cross(ab, ap))
    area_cap = _norm3(_cross(ac, ap))
    area_cbp = _norm3(_cross(bc, bp))
    w = area_bap / area_bac
    v = area_cap / area_bac
    u = area_cbp / area_bac
    denom = jnp.maximum(jnp.abs(u) + jnp.abs(v) + jnp.abs(w), 1e-12)
    u = u / denom
    v = v / denom
    w = w / denom

    fv = fv_ref[...].reshape(9, 8, _PB // 8)
    px = u * fv[0] + v * fv[3] + w * fv[6]
    py = u * fv[1] + v * fv[4] + w * fv[7]
    pz = u * fv[2] + v * fv[5] + w * fv[8]
    p3d_ref[...] = jnp.concatenate(
        [px[None], py[None], pz[None]], axis=0)[None]

    fvi = fvi_ref[...].reshape(3, _PB)
    vid = lax.broadcasted_iota(jnp.int32, (_VP, _PB), 0)
    wmat = ((vid == fvi[0:1, :]).astype(jnp.float32)
            + (vid == fvi[1:2, :]).astype(jnp.float32)
            + (vid == fvi[2:3, :]).astype(jnp.float32)).astype(jnp.bfloat16)

    feat = feat_ref[...].reshape(_PB, -1).astype(jnp.bfloat16)
    part = lax.dot_general(wmat, feat, (((1,), (0,)), ((), ())),
                           preferred_element_type=jnp.float32)
    ones8 = jnp.ones((_PB, 8), jnp.bfloat16)
    cnt8 = lax.dot_general(wmat, ones8, (((1,), (0,)), ((), ())),
                           preferred_element_type=jnp.float32)
    pcb = jnp.broadcast_to(cnt8[:, 0:1], (_VP, 128))

    partv = part[:_NUM_VERTS]
    pcbv = pcb[:_NUM_VERTS]

    @pl.when(p == 0)
    def _():
        vlf_ref[...] = partv[None]
        cnt_s[...] = pcbv

    @pl.when(p != 0)
    def _():
        vlf_ref[...] = vlf_ref[...] + partv[None]
        cnt_s[...] = cnt_s[...] + pcbv

    @pl.when(p == nblk - 1)
    def _():
        cnt = jnp.maximum(cnt_s[:, 0:1], 1.0)
        vlf_ref[...] = vlf_ref[...] / cnt[None]


def _tcb_call(fvi, fverts, fvt3d, uvrows, local_feature):
    b, pdim, d = local_feature.shape
    nw, _, pb = fvi.shape
    n = nw * pb
    nblk = pdim // pb
    assert b * nblk == nw and pb == _PB
    grid = (b, nblk)
    body = functools.partial(_tcb_body, nblk)
    return pl.pallas_call(
        body,
        grid=grid,
        in_specs=[
            pl.BlockSpec((1, 3, pb), lambda bi, pi: (bi * nblk + pi, 0, 0)),
            pl.BlockSpec((1, 9, 8, pb // 8),
                         lambda bi, pi: (bi * nblk + pi, 0, 0, 0)),
            pl.BlockSpec((1, 9, 8, pb // 8),
                         lambda bi, pi: (bi * nblk + pi, 0, 0, 0)),
            pl.BlockSpec((1, 2, 8, pb // 8),
                         lambda bi, pi: (bi * nblk + pi, 0, 0, 0)),
            pl.BlockSpec((1, pb, d), lambda bi, pi: (bi, pi, 0)),
        ],
        out_specs=[
            pl.BlockSpec((1, 3, 8, pb // 8),
                         lambda bi, pi: (bi * nblk + pi, 0, 0, 0)),
            pl.BlockSpec((1, _NUM_VERTS, d), lambda bi, pi: (bi, 0, 0)),
        ],
        out_shape=[
            jax.ShapeDtypeStruct((nw, 3, 8, pb // 8), jnp.float32),
            jax.ShapeDtypeStruct((b, _NUM_VERTS, d), jnp.float32),
        ],
        scratch_shapes=[pltpu.VMEM((_NUM_VERTS, 128), jnp.float32)],
    )(fvi, fverts, fvt3d, uvrows, local_feature)


def kernel(uv, local_feature, verts, uv_verts, faces, face_inds):
    n = uv.shape[0]
    uvrows, idx_fat, vt3d, faces_t, verts_t = _tca_call(
        uv, uv_verts, faces, verts)
    fvi, fverts, fvt3d, uv8 = _sc_gather_call(
        idx_fat, uvrows, face_inds.reshape(-1), faces_t, verts_t, vt3d, n)
    p3d4, vlf = _tcb_call(fvi, fverts, fvt3d, uv8, local_feature)
    p3d = jnp.transpose(p3d4, (0, 2, 3, 1)).reshape(n, 3)
    return p3d, vlf
